# Initial kernel scaffold; baseline (speedup 1.0000x reference)
import jax
import jax.numpy as jnp
from jax.experimental import pallas as pl


def kernel(x, table):
    raise NotImplementedError("write your pallas kernel here")


# speedup vs baseline: 4.5069x; 4.5069x over previous
import functools

import jax
import jax.numpy as jnp
from jax import lax
from jax.experimental import pallas as pl
from jax.experimental.pallas import tpu as pltpu
from jax.experimental.pallas import tpu_sc as plsc

EMBED_DIM = 64
IDX_W = 128
G = 8


@functools.lru_cache(maxsize=None)
def _make_gather(num_rows128: int, vocab: int, embed_dim: int):
    info = plsc.get_sparse_core_info()
    nc, ns = info.num_cores, info.num_subcores
    nw = nc * ns
    assert num_rows128 % nw == 0
    rows_per_w = num_rows128 // nw

    mesh = plsc.VectorSubcoreMesh(core_axis_name="c", subcore_axis_name="s")

    @functools.partial(
        pl.kernel,
        mesh=mesh,
        out_type=jax.ShapeDtypeStruct((num_rows128, IDX_W, embed_dim),
                                      jnp.float32),
        compiler_params=pltpu.CompilerParams(use_tc_tiling_on_sc=False),
        scratch_types=[
            pltpu.VMEM((IDX_W,), jnp.int32),
            pltpu.VMEM((IDX_W, embed_dim), jnp.float32),
            pltpu.SemaphoreType.DMA,
        ],
    )
    def gather_kernel(table_hbm, idx_hbm, out_hbm, idx_v, rows_v, sem):
        wid = lax.axis_index("s") * nc + lax.axis_index("c")
        base = wid * rows_per_w

        def chunk_body(c, carry):
            row0 = base + c
            pltpu.sync_copy(idx_hbm.at[row0], idx_v)
            pltpu.async_copy(table_hbm.at[idx_v], rows_v, sem).wait()
            pltpu.sync_copy(rows_v, out_hbm.at[row0])
            return carry

        lax.fori_loop(0, rows_per_w, chunk_body, 0)

    return gather_kernel


def kernel(x, table):
    vocab, embed_dim = table.shape
    b = x.size
    idx = x.reshape(b // IDX_W, IDX_W).astype(jnp.int32)
    fn = _make_gather(b // IDX_W, vocab, embed_dim)
    out = fn(table, idx)
    return out.reshape(x.shape + (embed_dim,))


# speedup vs baseline: 5.0933x; 1.1301x over previous
import functools

import jax
import jax.numpy as jnp
from jax import lax
from jax.experimental import pallas as pl
from jax.experimental.pallas import tpu as pltpu
from jax.experimental.pallas import tpu_sc as plsc

EMBED_DIM = 64
IDX_W = 128
G = 8


@functools.lru_cache(maxsize=None)
def _make_gather(num_rows128: int, vocab: int, embed_dim: int):
    info = plsc.get_sparse_core_info()
    nc, ns = info.num_cores, info.num_subcores
    nw = nc * ns
    assert num_rows128 % (nw * G) == 0
    rows_per_w = num_rows128 // nw
    num_chunks = rows_per_w // G

    mesh = plsc.VectorSubcoreMesh(core_axis_name="c", subcore_axis_name="s")

    @functools.partial(
        pl.kernel,
        mesh=mesh,
        out_type=jax.ShapeDtypeStruct((num_rows128, IDX_W, embed_dim),
                                      jnp.float32),
        compiler_params=pltpu.CompilerParams(use_tc_tiling_on_sc=False),
        scratch_types=(
            [pltpu.VMEM((IDX_W,), jnp.int32) for _ in range(G)]
            + [pltpu.VMEM((G, IDX_W, embed_dim), jnp.float32),
               pltpu.SemaphoreType.DMA]
        ),
    )
    def gather_kernel(table_hbm, idx_hbm, out_hbm, *scratch):
        idx_vs = scratch[:G]
        rows_v, sem = scratch[G], scratch[G + 1]
        wid = lax.axis_index("s") * nc + lax.axis_index("c")
        base = wid * rows_per_w

        def chunk_body(c, carry):
            row0 = base + c * G
            for j in range(G):
                pltpu.sync_copy(idx_hbm.at[row0 + j], idx_vs[j])
            descs = [
                pltpu.async_copy(table_hbm.at[idx_vs[j]], rows_v.at[j], sem)
                for j in range(G)
            ]
            for d in descs:
                d.wait()
            pltpu.sync_copy(rows_v, out_hbm.at[pl.ds(row0, G)])
            return carry

        lax.fori_loop(0, num_chunks, chunk_body, 0)

    return gather_kernel


def kernel(x, table):
    vocab, embed_dim = table.shape
    b = x.size
    idx = x.reshape(b // IDX_W, IDX_W).astype(jnp.int32)
    fn = _make_gather(b // IDX_W, vocab, embed_dim)
    out = fn(table, idx)
    return out.reshape(x.shape + (embed_dim,))


# speedup vs baseline: 5.6542x; 1.1101x over previous
import functools

import jax
import jax.numpy as jnp
from jax import lax
from jax.experimental import pallas as pl
from jax.experimental.pallas import tpu as pltpu
from jax.experimental.pallas import tpu_sc as plsc

EMBED_DIM = 64
IDX_W = 128
G = 5


@functools.lru_cache(maxsize=None)
def _make_gather(num_rows128: int, vocab: int, embed_dim: int):
    info = plsc.get_sparse_core_info()
    nc, ns = info.num_cores, info.num_subcores
    nw = nc * ns
    assert num_rows128 % (nw * G) == 0
    rows_per_w = num_rows128 // nw
    num_chunks = rows_per_w // G
    assert num_chunks % 2 == 0

    mesh = plsc.VectorSubcoreMesh(core_axis_name="c", subcore_axis_name="s")

    @functools.partial(
        pl.kernel,
        mesh=mesh,
        out_type=jax.ShapeDtypeStruct((num_rows128, IDX_W, embed_dim),
                                      jnp.float32),
        compiler_params=pltpu.CompilerParams(use_tc_tiling_on_sc=False),
        scratch_types=(
            [pltpu.VMEM((IDX_W,), jnp.int32) for _ in range(2 * G)]
            + [pltpu.VMEM((G, IDX_W, embed_dim), jnp.float32) for _ in range(2)]
            + [pltpu.SemaphoreType.DMA for _ in range(6)]
        ),
    )
    def gather_kernel(table_hbm, idx_hbm, out_hbm, *scratch):
        idx_vs = (scratch[:G], scratch[G:2 * G])
        rows_vs = scratch[2 * G:2 * G + 2]
        gsem = scratch[2 * G + 2:2 * G + 4]
        isem = scratch[2 * G + 4:2 * G + 6]
        osem = scratch[2 * G + 6:2 * G + 8]

        wid = lax.axis_index("s") * nc + lax.axis_index("c")
        base = wid * rows_per_w

        def fire_idx(b, c):
            for j in range(G):
                pltpu.async_copy(idx_hbm.at[base + c * G + j],
                                 idx_vs[b][j], isem[b])

        def drain_idx(b, c):
            for j in range(G):
                pltpu.make_async_copy(idx_hbm.at[base + c * G + j],
                                      idx_vs[b][j], isem[b]).wait()

        def fire_gathers(b):
            for j in range(G):
                pltpu.async_copy(table_hbm.at[idx_vs[b][j]],
                                 rows_vs[b].at[j], gsem[b])

        def drain_gathers(b):
            for j in range(G):
                pltpu.make_async_copy(table_hbm.at[idx_vs[b][j]],
                                      rows_vs[b].at[j], gsem[b]).wait()

        def fire_write(b, c):
            pltpu.async_copy(rows_vs[b],
                             out_hbm.at[pl.ds(base + c * G, G)], osem[b])

        def drain_write(b, c):
            pltpu.make_async_copy(rows_vs[b],
                                  out_hbm.at[pl.ds(base + c * G, G)],
                                  osem[b]).wait()

        for j in range(G):
            pltpu.sync_copy(idx_hbm.at[base + j], idx_vs[0][j])
        fire_gathers(0)
        fire_idx(1, 1)

        def pair_body(s, carry):
            for half in range(2):
                c = 2 * s + half
                b = half

                @pl.when(c + 1 < num_chunks)
                def _():
                    drain_idx(1 - b, c + 1)

                @pl.when(c > 0)
                def _():
                    drain_write(1 - b, c - 1)

                @pl.when(c + 1 < num_chunks)
                def _():
                    fire_gathers(1 - b)

                drain_gathers(b)
                fire_write(b, c)

                @pl.when(c + 2 < num_chunks)
                def _():
                    fire_idx(b, c + 2)

            return carry

        lax.fori_loop(0, num_chunks // 2, pair_body, 0)
        drain_write(1, num_chunks - 1)

    return gather_kernel


def kernel(x, table):
    vocab, embed_dim = table.shape
    b = x.size
    idx = x.reshape(b // IDX_W, IDX_W).astype(jnp.int32)
    fn = _make_gather(b // IDX_W, vocab, embed_dim)
    out = fn(table, idx)
    return out.reshape(x.shape + (embed_dim,))
